# baseline (speedup 1.0000x reference)
import functools

import jax
import jax.numpy as jnp
from jax import lax
from jax.experimental import pallas as pl
from jax.experimental.pallas import tpu as pltpu
from jax.experimental.pallas import tpu_sc as plsc

N = 10000
E = 320000
D = 128
C = 7
L = 16
NC, NS = 2, 16
NW = NC * NS
EPT = E // NW
CH = 128
KG = 16
NCH = EPT // CH
NG_FULL = NCH // KG
REM = NCH - NG_FULL * KG
TAIL = EPT - NCH * CH
N_PAD = 10240
RPT = N_PAD // NS
BN = 512

_SC_PARAMS = pltpu.CompilerParams(use_tc_tiling_on_sc=False)
_SC_PARAMS_NL = pltpu.CompilerParams(use_tc_tiling_on_sc=False,
                                     needs_layout_passes=False)
_MESH = plsc.VectorSubcoreMesh(core_axis_name="c", subcore_axis_name="s",
                               num_cores=NC, num_subcores=NS)


def _zero_acc_slab(rows_v, acc, sid):
  @pl.loop(0, CH)
  def _(i):
    rows_v[0, i, :] = jnp.zeros((L,), jnp.float32)

  @pl.loop(0, RPT // CH)
  def _(k):
    pltpu.sync_copy(rows_v.at[0], acc.at[pl.ds(sid * RPT + k * CH, CH)])


def _edge_sweep(gather_src, src_v, dst_v, rows_v, acc, sem_g, sem_s):
  def do_group(cbase, k_count):
    if gather_src is not None:
      gds = [pltpu.async_copy(
          gather_src.at[src_v.at[pl.ds((cbase + k) * CH, CH)]],
          rows_v.at[k], sem_g) for k in range(k_count)]
      for d in gds:
        d.wait()
    sds = [pltpu.async_copy(
        rows_v.at[k if gather_src is not None else 0],
        acc.at[dst_v.at[pl.ds((cbase + k) * CH, CH)]],
        sem_s, add=True) for k in range(k_count)]
    for d in sds:
      d.wait()

  @pl.loop(0, NG_FULL)
  def _(g):
    do_group(g * KG, KG)

  do_group(NG_FULL * KG, REM)

  if gather_src is not None:
    pltpu.async_copy(
        gather_src.at[src_v.at[pl.ds(NCH * CH, TAIL)]],
        rows_v.at[0, pl.ds(0, TAIL)], sem_g).wait()
  pltpu.async_copy(
      rows_v.at[0, pl.ds(0, TAIL)],
      acc.at[dst_v.at[pl.ds(NCH * CH, TAIL)]],
      sem_s, add=True).wait()


@functools.partial(
    pl.kernel,
    out_type=jax.ShapeDtypeStruct((NC, N_PAD, L), jnp.float32),
    mesh=_MESH,
    compiler_params=_SC_PARAMS,
    scratch_types=[
        pltpu.VMEM((EPT,), jnp.int32),
        pltpu.VMEM((KG, CH, L), jnp.float32),
        pltpu.VMEM_SHARED((N_PAD, L), jnp.float32),
        pltpu.SemaphoreType.DMA,
        pltpu.SemaphoreType.DMA,
    ],
)
def _sc_deg(ei_hbm, out_hbm, dst_v, rows_v, acc, sem_g, sem_s):
  cid = lax.axis_index("c")
  sid = lax.axis_index("s")
  wid = cid * NS + sid
  ds_dst = pltpu.async_copy(ei_hbm.at[1, pl.ds(wid * EPT, EPT)], dst_v, sem_s)

  _zero_acc_slab(rows_v, acc, sid)

  @pl.loop(0, CH)
  def _(i):
    rows_v[0, i, :] = jnp.ones((L,), jnp.float32)

  ds_dst.wait()
  plsc.subcore_barrier()
  _edge_sweep(None, None, dst_v, rows_v, acc, sem_g, sem_s)
  plsc.subcore_barrier()
  slab = pl.ds(sid * RPT, RPT)
  pltpu.sync_copy(acc.at[slab], out_hbm.at[cid, slab])


def _rsqrt16(x):
  yi = jnp.int32(0x5F3759DF) - lax.shift_right_logical(
      lax.bitcast_convert_type(x, jnp.int32), 1)
  y = lax.bitcast_convert_type(yi, jnp.float32)
  for _ in range(3):
    y = y * (1.5 - 0.5 * x * y * y)
  return y


def _make_sc_hop(first: bool):
  agg_t = jax.ShapeDtypeStruct((NC, N_PAD, L), jnp.float32)
  row_t = jax.ShapeDtypeStruct((N_PAD, L), jnp.float32)
  out_type = [agg_t, row_t, row_t] if first else [agg_t, row_t]

  @functools.partial(
      pl.kernel,
      out_type=out_type,
      mesh=_MESH,
      compiler_params=_SC_PARAMS,
      scratch_types=[
          pltpu.VMEM((EPT,), jnp.int32),
          pltpu.VMEM((EPT,), jnp.int32),
          pltpu.VMEM((KG, CH, L), jnp.float32),
          pltpu.VMEM((3 if first else 4, RPT, L), jnp.float32),
          pltpu.VMEM_SHARED((N_PAD, L), jnp.float32),
          pltpu.VMEM_SHARED((N_PAD, L), jnp.float32),
          pltpu.SemaphoreType.DMA,
          pltpu.SemaphoreType.DMA,
      ],
  )
  def sc_hop(*refs):
    if first:
      (d0_hbm, d1_hbm, z_hbm, ei_hbm, agg_out, u_out, inv_out,
       src_v, dst_v, rows_v, work_v, u_spm, acc, sem_g, sem_s) = refs
    else:
      (a0_hbm, a1_hbm, u_hbm, inv_hbm, ei_hbm, agg_out, u_out,
       src_v, dst_v, rows_v, work_v, u_spm, acc, sem_g, sem_s) = refs
    cid = lax.axis_index("c")
    sid = lax.axis_index("s")
    wid = cid * NS + sid
    slab = pl.ds(sid * RPT, RPT)

    ds_src = pltpu.async_copy(ei_hbm.at[0, pl.ds(wid * EPT, EPT)], src_v,
                              sem_g)
    ds_dst = pltpu.async_copy(ei_hbm.at[1, pl.ds(wid * EPT, EPT)], dst_v,
                              sem_s)

    if first:
      pltpu.sync_copy(d0_hbm.at[slab], work_v.at[0])
      pltpu.sync_copy(d1_hbm.at[slab], work_v.at[1])
      pltpu.sync_copy(z_hbm.at[slab], work_v.at[2])

      @pl.loop(0, RPT)
      def _(i):
        deg = work_v[0, i, :] + work_v[1, i, :] + 1.0
        work_v[0, i, :] = _rsqrt16(deg) * work_v[2, i, :]
        work_v[1, i, :] = 1.0 / deg
    else:
      pltpu.sync_copy(a0_hbm.at[slab], work_v.at[0])
      pltpu.sync_copy(a1_hbm.at[slab], work_v.at[1])
      pltpu.sync_copy(u_hbm.at[slab], work_v.at[2])
      pltpu.sync_copy(inv_hbm.at[slab], work_v.at[3])

      @pl.loop(0, RPT)
      def _(i):
        work_v[0, i, :] = work_v[3, i, :] * (
            work_v[0, i, :] + work_v[1, i, :] + work_v[2, i, :])

    pltpu.sync_copy(work_v.at[0], u_spm.at[slab])

    @pl.when(cid == 0)
    def _():
      pltpu.sync_copy(work_v.at[0], u_out.at[slab])
      if first:
        pltpu.sync_copy(work_v.at[1], inv_out.at[slab])

    _zero_acc_slab(rows_v, acc, sid)
    ds_src.wait()
    ds_dst.wait()
    plsc.subcore_barrier()
    _edge_sweep(u_spm, src_v, dst_v, rows_v, acc, sem_g, sem_s)
    plsc.subcore_barrier()
    pltpu.sync_copy(acc.at[slab], agg_out.at[cid, slab])

  return sc_hop


_sc_hop1 = _make_sc_hop(first=True)
_sc_hop2 = _make_sc_hop(first=False)


def _row_specs(n):
  return [pl.BlockSpec((BN, L), lambda i: (i, 0)) for _ in range(n)]


def _tc_proj(x, wp):
  def body(x_ref, w_ref, z_ref):
    z_ref[...] = jnp.dot(x_ref[...], w_ref[...],
                         preferred_element_type=jnp.float32)

  return pl.pallas_call(
      body,
      grid=(N_PAD // BN,),
      in_specs=[pl.BlockSpec((BN, D), lambda i: (i, 0)),
                pl.BlockSpec((D, L), lambda i: (0, 0))],
      out_specs=_row_specs(1)[0],
      out_shape=jax.ShapeDtypeStruct((N_PAD, L), jnp.float32),
  )(x, wp)


def _tc_post(a0, a1, u1, d0, d1, b16):
  def body(a0_ref, a1_ref, u1_ref, d0_ref, d1_ref, b_ref, o_ref):
    dis = lax.rsqrt(d0_ref[...] + d1_ref[...] + 1.0)
    h2 = dis * (a0_ref[...] + a1_ref[...] + u1_ref[...])
    logits = h2 + b_ref[...]
    col = lax.broadcasted_iota(jnp.int32, (BN, L), 1)
    valid = col < C
    masked = jnp.where(valid, logits, jnp.float32(-1e30))
    m = jnp.max(masked, axis=1, keepdims=True)
    s = jnp.sum(jnp.where(valid, jnp.exp(logits - m), 0.0), axis=1,
                keepdims=True)
    o_ref[...] = logits - m - jnp.log(s)

  return pl.pallas_call(
      body,
      grid=(N_PAD // BN,),
      in_specs=_row_specs(5) + [pl.BlockSpec((1, L), lambda i: (0, 0))],
      out_specs=_row_specs(1)[0],
      out_shape=jax.ShapeDtypeStruct((N_PAD, L), jnp.float32),
  )(a0, a1, u1, d0, d1, b16)


def kernel(x, edge_index, W, b):
  wp = jnp.pad(W.T.astype(jnp.float32), ((0, 0), (0, L - C)))
  b16 = jnp.pad(b.astype(jnp.float32), (0, L - C)).reshape(1, L)

  z = _tc_proj(x, wp)
  degp = _sc_deg(edge_index)
  a1, u0, inv = _sc_hop1(degp[0], degp[1], z, edge_index)
  a2, u1 = _sc_hop2(a1[0], a1[1], u0, inv, edge_index)
  out = _tc_post(a2[0], a2[1], u1, degp[0], degp[1], b16)
  return out[:N, :C]


# speedup vs baseline: 1.0553x; 1.0553x over previous
import functools

import jax
import jax.numpy as jnp
from jax import lax
from jax.experimental import pallas as pl
from jax.experimental.pallas import tpu as pltpu
from jax.experimental.pallas import tpu_sc as plsc

N = 10000
E = 320000
D = 128
C = 7
L = 16
NC, NS = 2, 16
NW = NC * NS
EPT = E // NW
CH = 128
NCH = EPT // CH
TAIL = EPT - NCH * CH
KG = 16
KP = 13
NPAIR = NCH // (2 * KP)
N_PAD = 10240
RPT = N_PAD // NS
BN = 512

_SC_PARAMS = pltpu.CompilerParams(use_tc_tiling_on_sc=False)
_MESH = plsc.VectorSubcoreMesh(core_axis_name="c", subcore_axis_name="s",
                               num_cores=NC, num_subcores=NS)


def _zero_acc_slab(rows_v, acc, sid):
  @pl.loop(0, CH)
  def _(i):
    rows_v[0, i, :] = jnp.zeros((L,), jnp.float32)

  @pl.loop(0, RPT // CH)
  def _(k):
    pltpu.sync_copy(rows_v.at[0], acc.at[pl.ds(sid * RPT + k * CH, CH)])


def _sweep_tail(u_spm, src_v, dst_v, rows_v, acc, sem_g, sem_s):
  if u_spm is not None:
    pltpu.async_copy(
        u_spm.at[src_v.at[pl.ds(NCH * CH, TAIL)]],
        rows_v.at[0, pl.ds(0, TAIL)], sem_g).wait()
  pltpu.async_copy(
      rows_v.at[0, pl.ds(0, TAIL)],
      acc.at[dst_v.at[pl.ds(NCH * CH, TAIL)]],
      sem_s, add=True).wait()


def _sweep_pipelined(u_spm, src_v, dst_v, rows_v, acc, sem_g, sem_s):
  def fire_g(c0, off):
    for k in range(KP):
      pltpu.async_copy(u_spm.at[src_v.at[pl.ds((c0 + k) * CH, CH)]],
                       rows_v.at[off + k], sem_g)

  def drain_g(c0, off):
    for k in range(KP):
      pltpu.make_async_copy(u_spm.at[src_v.at[pl.ds((c0 + k) * CH, CH)]],
                            rows_v.at[off + k], sem_g).wait()

  def fire_s(c0, off):
    for k in range(KP):
      pltpu.async_copy(rows_v.at[off + k],
                       acc.at[dst_v.at[pl.ds((c0 + k) * CH, CH)]],
                       sem_s, add=True)

  def drain_s(c0, off):
    for k in range(KP):
      pltpu.make_async_copy(rows_v.at[off + k],
                            acc.at[dst_v.at[pl.ds((c0 + k) * CH, CH)]],
                            sem_s).wait()

  fire_g(0, 0)

  @pl.loop(0, NPAIR)
  def _(p):
    ca = 2 * p * KP
    cb = ca + KP
    drain_g(ca, 0)
    fire_s(ca, 0)
    fire_g(cb, KP)
    drain_s(ca, 0)
    drain_g(cb, KP)
    fire_s(cb, KP)

    @pl.when(p < NPAIR - 1)
    def _():
      fire_g(cb + KP, 0)

    drain_s(cb, KP)

  _sweep_tail(u_spm, src_v, dst_v, rows_v, acc, sem_g, sem_s)


@functools.partial(
    pl.kernel,
    out_type=jax.ShapeDtypeStruct((NC, N_PAD, L), jnp.float32),
    mesh=_MESH,
    compiler_params=_SC_PARAMS,
    scratch_types=[
        pltpu.VMEM((EPT,), jnp.int32),
        pltpu.VMEM((1, CH, L), jnp.float32),
        pltpu.VMEM_SHARED((N_PAD, L), jnp.float32),
        pltpu.SemaphoreType.DMA,
        pltpu.SemaphoreType.DMA,
    ],
)
def _sc_deg(ei_hbm, out_hbm, dst_v, rows_v, acc, sem_g, sem_s):
  cid = lax.axis_index("c")
  sid = lax.axis_index("s")
  wid = cid * NS + sid
  ds_dst = pltpu.async_copy(ei_hbm.at[1, pl.ds(wid * EPT, EPT)], dst_v, sem_s)

  _zero_acc_slab(rows_v, acc, sid)

  @pl.loop(0, CH)
  def _(i):
    rows_v[0, i, :] = jnp.ones((L,), jnp.float32)

  ds_dst.wait()
  plsc.subcore_barrier()

  @pl.loop(0, NCH // KG)
  def _(g):
    sds = [pltpu.async_copy(
        rows_v.at[0],
        acc.at[dst_v.at[pl.ds((g * KG + k) * CH, CH)]],
        sem_s, add=True) for k in range(KG)]
    for d in sds:
      d.wait()

  rem = NCH - (NCH // KG) * KG
  sds = [pltpu.async_copy(
      rows_v.at[0],
      acc.at[dst_v.at[pl.ds(((NCH // KG) * KG + k) * CH, CH)]],
      sem_s, add=True) for k in range(rem)]
  for d in sds:
    d.wait()

  _sweep_tail(None, None, dst_v, rows_v, acc, sem_g, sem_s)
  plsc.subcore_barrier()
  slab = pl.ds(sid * RPT, RPT)
  pltpu.sync_copy(acc.at[slab], out_hbm.at[cid, slab])


def _make_sc_hop(combine: bool):
  agg_t = jax.ShapeDtypeStruct((NC, N_PAD, L), jnp.float32)
  out_type = [agg_t, jax.ShapeDtypeStruct((N_PAD, L), jnp.float32)] \
      if combine else agg_t
  HRPT = RPT // 2
  scratch = [
      pltpu.VMEM((EPT,), jnp.int32),
      pltpu.VMEM((EPT,), jnp.int32),
      pltpu.VMEM((2 * KP, CH, L), jnp.float32),
  ]
  if combine:
    scratch.append(pltpu.VMEM((4, HRPT, L), jnp.float32))
  scratch += [
      pltpu.VMEM_SHARED((N_PAD, L), jnp.float32),
      pltpu.VMEM_SHARED((N_PAD, L), jnp.float32),
      pltpu.SemaphoreType.DMA,
      pltpu.SemaphoreType.DMA,
  ]

  @functools.partial(
      pl.kernel,
      out_type=out_type,
      mesh=_MESH,
      compiler_params=_SC_PARAMS,
      scratch_types=scratch,
  )
  def sc_hop(*refs):
    if combine:
      (a0_hbm, a1_hbm, u_hbm, inv_hbm, ei_hbm, agg_out, u_out,
       src_v, dst_v, rows_v, work_v, u_spm, acc, sem_g, sem_s) = refs
    else:
      (u_hbm, ei_hbm, agg_out,
       src_v, dst_v, rows_v, u_spm, acc, sem_g, sem_s) = refs
    cid = lax.axis_index("c")
    sid = lax.axis_index("s")
    wid = cid * NS + sid
    slab = pl.ds(sid * RPT, RPT)

    ds_src = pltpu.async_copy(ei_hbm.at[0, pl.ds(wid * EPT, EPT)], src_v,
                              sem_g)
    ds_dst = pltpu.async_copy(ei_hbm.at[1, pl.ds(wid * EPT, EPT)], dst_v,
                              sem_s)

    if combine:
      for h in range(2):
        hslab = pl.ds(sid * RPT + h * HRPT, HRPT)
        pltpu.sync_copy(a0_hbm.at[hslab], work_v.at[0])
        pltpu.sync_copy(a1_hbm.at[hslab], work_v.at[1])
        pltpu.sync_copy(u_hbm.at[hslab], work_v.at[2])
        pltpu.sync_copy(inv_hbm.at[hslab], work_v.at[3])

        @pl.loop(0, HRPT)
        def _(i):
          work_v[0, i, :] = work_v[3, i, :] * (
              work_v[0, i, :] + work_v[1, i, :] + work_v[2, i, :])

        pltpu.sync_copy(work_v.at[0], u_spm.at[hslab])

        @pl.when(cid == 0)
        def _():
          pltpu.sync_copy(work_v.at[0], u_out.at[hslab])
    else:
      pltpu.sync_copy(u_hbm.at[slab], u_spm.at[slab])

    _zero_acc_slab(rows_v, acc, sid)
    ds_src.wait()
    ds_dst.wait()
    plsc.subcore_barrier()
    _sweep_pipelined(u_spm, src_v, dst_v, rows_v, acc, sem_g, sem_s)
    plsc.subcore_barrier()
    pltpu.sync_copy(acc.at[slab], agg_out.at[cid, slab])

  return sc_hop


_sc_hop = _make_sc_hop(combine=False)
_sc_hop_fused = _make_sc_hop(combine=True)


def _row_specs(n):
  return [pl.BlockSpec((BN, L), lambda i: (i, 0)) for _ in range(n)]


def _tc_pre(x, wp, d0, d1):
  def body(x_ref, w_ref, d0_ref, d1_ref, u0_ref, inv_ref, dis_ref):
    deg = d0_ref[...] + d1_ref[...] + 1.0
    inv = 1.0 / deg
    dis = lax.rsqrt(deg)
    z = jnp.dot(x_ref[...], w_ref[...], preferred_element_type=jnp.float32)
    u0_ref[...] = dis * z
    inv_ref[...] = inv
    dis_ref[...] = dis

  return pl.pallas_call(
      body,
      grid=(N_PAD // BN,),
      in_specs=[pl.BlockSpec((BN, D), lambda i: (i, 0)),
                pl.BlockSpec((D, L), lambda i: (0, 0))] + _row_specs(2),
      out_specs=_row_specs(3),
      out_shape=[jax.ShapeDtypeStruct((N_PAD, L), jnp.float32)] * 3,
  )(x, wp, d0, d1)


def _tc_post(a0, a1, u1, dis, b16):
  def body(a0_ref, a1_ref, u1_ref, dis_ref, b_ref, o_ref):
    h2 = dis_ref[...] * (a0_ref[...] + a1_ref[...] + u1_ref[...])
    logits = h2 + b_ref[...]
    col = lax.broadcasted_iota(jnp.int32, (BN, L), 1)
    valid = col < C
    masked = jnp.where(valid, logits, jnp.float32(-1e30))
    m = jnp.max(masked, axis=1, keepdims=True)
    s = jnp.sum(jnp.where(valid, jnp.exp(logits - m), 0.0), axis=1,
                keepdims=True)
    o_ref[...] = logits - m - jnp.log(s)

  return pl.pallas_call(
      body,
      grid=(N_PAD // BN,),
      in_specs=_row_specs(3) + [pl.BlockSpec((BN, L), lambda i: (i, 0)),
                                pl.BlockSpec((1, L), lambda i: (0, 0))],
      out_specs=_row_specs(1)[0],
      out_shape=jax.ShapeDtypeStruct((N_PAD, L), jnp.float32),
  )(a0, a1, u1, dis, b16)


def kernel(x, edge_index, W, b):
  wp = jnp.pad(W.T.astype(jnp.float32), ((0, 0), (0, L - C)))
  b16 = jnp.pad(b.astype(jnp.float32), (0, L - C)).reshape(1, L)

  degp = _sc_deg(edge_index)
  u0, inv, dis = _tc_pre(x, wp, degp[0], degp[1])
  a1 = _sc_hop(u0, edge_index)
  a2, u1 = _sc_hop_fused(a1[0], a1[1], u0, inv, edge_index)
  out = _tc_post(a2[0], a2[1], u1, dis, b16)
  return out[:N, :C]


# speedup vs baseline: 1.1432x; 1.0833x over previous
import functools

import jax
import jax.numpy as jnp
from jax import lax
from jax.experimental import pallas as pl
from jax.experimental.pallas import tpu as pltpu
from jax.experimental.pallas import tpu_sc as plsc

N = 10000
E = 320000
D = 128
C = 7
L = 16
NC, NS = 2, 16
NW = NC * NS
EPT = E // NW
CH = 128
NCH = EPT // CH
TAIL = EPT - NCH * CH
KG = 16
KP = 13
NPAIR = NCH // (2 * KP)
N_PAD = 10240
RPT = N_PAD // NS
BN = N_PAD

_SC_PARAMS = pltpu.CompilerParams(use_tc_tiling_on_sc=False)
_MESH = plsc.VectorSubcoreMesh(core_axis_name="c", subcore_axis_name="s",
                               num_cores=NC, num_subcores=NS)


def _zero_acc_slab(rows_v, acc, sid):
  @pl.loop(0, CH)
  def _(i):
    rows_v[0, i, :] = jnp.zeros((L,), jnp.float32)

  @pl.loop(0, RPT // CH)
  def _(k):
    pltpu.sync_copy(rows_v.at[0], acc.at[pl.ds(sid * RPT + k * CH, CH)])


def _sweep_tail(u_spm, src_v, dst_v, rows_v, acc, sem_g, sem_s):
  if u_spm is not None:
    pltpu.async_copy(
        u_spm.at[src_v.at[pl.ds(NCH * CH, TAIL)]],
        rows_v.at[0, pl.ds(0, TAIL)], sem_g).wait()
  pltpu.async_copy(
      rows_v.at[0, pl.ds(0, TAIL)],
      acc.at[dst_v.at[pl.ds(NCH * CH, TAIL)]],
      sem_s, add=True).wait()


def _sweep_pipelined(u_spm, src_v, dst_v, rows_v, acc, sem_g, sem_s):
  def fire_g(c0, off):
    for k in range(KP):
      pltpu.async_copy(u_spm.at[src_v.at[pl.ds((c0 + k) * CH, CH)]],
                       rows_v.at[off + k], sem_g)

  def drain_g(c0, off):
    for k in range(KP):
      pltpu.make_async_copy(u_spm.at[src_v.at[pl.ds((c0 + k) * CH, CH)]],
                            rows_v.at[off + k], sem_g).wait()

  def fire_s(c0, off):
    for k in range(KP):
      pltpu.async_copy(rows_v.at[off + k],
                       acc.at[dst_v.at[pl.ds((c0 + k) * CH, CH)]],
                       sem_s, add=True)

  def drain_s(c0, off):
    for k in range(KP):
      pltpu.make_async_copy(rows_v.at[off + k],
                            acc.at[dst_v.at[pl.ds((c0 + k) * CH, CH)]],
                            sem_s).wait()

  fire_g(0, 0)

  @pl.loop(0, NPAIR)
  def _(p):
    ca = 2 * p * KP
    cb = ca + KP
    drain_g(ca, 0)
    fire_s(ca, 0)
    fire_g(cb, KP)
    drain_s(ca, 0)
    drain_g(cb, KP)
    fire_s(cb, KP)

    @pl.when(p < NPAIR - 1)
    def _():
      fire_g(cb + KP, 0)

    drain_s(cb, KP)

  _sweep_tail(u_spm, src_v, dst_v, rows_v, acc, sem_g, sem_s)


@functools.partial(
    pl.kernel,
    out_type=jax.ShapeDtypeStruct((NC, N_PAD, L), jnp.float32),
    mesh=_MESH,
    compiler_params=_SC_PARAMS,
    scratch_types=[
        pltpu.VMEM((EPT,), jnp.int32),
        pltpu.VMEM((1, CH, L), jnp.float32),
        pltpu.VMEM_SHARED((N_PAD, L), jnp.float32),
        pltpu.SemaphoreType.DMA,
        pltpu.SemaphoreType.DMA,
    ],
)
def _sc_deg(ei_hbm, out_hbm, dst_v, rows_v, acc, sem_g, sem_s):
  cid = lax.axis_index("c")
  sid = lax.axis_index("s")
  wid = cid * NS + sid
  ds_dst = pltpu.async_copy(ei_hbm.at[1, pl.ds(wid * EPT, EPT)], dst_v, sem_s)

  _zero_acc_slab(rows_v, acc, sid)

  @pl.loop(0, CH)
  def _(i):
    rows_v[0, i, :] = jnp.ones((L,), jnp.float32)

  ds_dst.wait()
  plsc.subcore_barrier()

  @pl.loop(0, NCH // KG)
  def _(g):
    sds = [pltpu.async_copy(
        rows_v.at[0],
        acc.at[dst_v.at[pl.ds((g * KG + k) * CH, CH)]],
        sem_s, add=True) for k in range(KG)]
    for d in sds:
      d.wait()

  rem = NCH - (NCH // KG) * KG
  sds = [pltpu.async_copy(
      rows_v.at[0],
      acc.at[dst_v.at[pl.ds(((NCH // KG) * KG + k) * CH, CH)]],
      sem_s, add=True) for k in range(rem)]
  for d in sds:
    d.wait()

  _sweep_tail(None, None, dst_v, rows_v, acc, sem_g, sem_s)
  plsc.subcore_barrier()
  slab = pl.ds(sid * RPT, RPT)
  pltpu.sync_copy(acc.at[slab], out_hbm.at[cid, slab])


def _make_sc_hop(combine: bool):
  agg_t = jax.ShapeDtypeStruct((NC, N_PAD, L), jnp.float32)
  out_type = [agg_t, jax.ShapeDtypeStruct((N_PAD, L), jnp.float32)] \
      if combine else agg_t
  HRPT = RPT // 2
  scratch = [
      pltpu.VMEM((EPT,), jnp.int32),
      pltpu.VMEM((EPT,), jnp.int32),
      pltpu.VMEM((2 * KP, CH, L), jnp.float32),
  ]
  if combine:
    scratch.append(pltpu.VMEM((4, HRPT, L), jnp.float32))
  scratch += [
      pltpu.VMEM_SHARED((N_PAD, L), jnp.float32),
      pltpu.VMEM_SHARED((N_PAD, L), jnp.float32),
      pltpu.SemaphoreType.DMA,
      pltpu.SemaphoreType.DMA,
  ]

  @functools.partial(
      pl.kernel,
      out_type=out_type,
      mesh=_MESH,
      compiler_params=_SC_PARAMS,
      scratch_types=scratch,
  )
  def sc_hop(*refs):
    if combine:
      (a0_hbm, a1_hbm, u_hbm, inv_hbm, ei_hbm, agg_out, u_out,
       src_v, dst_v, rows_v, work_v, u_spm, acc, sem_g, sem_s) = refs
    else:
      (u_hbm, ei_hbm, agg_out,
       src_v, dst_v, rows_v, u_spm, acc, sem_g, sem_s) = refs
    cid = lax.axis_index("c")
    sid = lax.axis_index("s")
    wid = cid * NS + sid
    slab = pl.ds(sid * RPT, RPT)

    ds_src = pltpu.async_copy(ei_hbm.at[0, pl.ds(wid * EPT, EPT)], src_v,
                              sem_g)
    ds_dst = pltpu.async_copy(ei_hbm.at[1, pl.ds(wid * EPT, EPT)], dst_v,
                              sem_s)

    if combine:
      for h in range(2):
        hslab = pl.ds(sid * RPT + h * HRPT, HRPT)
        pltpu.sync_copy(a0_hbm.at[hslab], work_v.at[0])
        pltpu.sync_copy(a1_hbm.at[hslab], work_v.at[1])
        pltpu.sync_copy(u_hbm.at[hslab], work_v.at[2])
        pltpu.sync_copy(inv_hbm.at[hslab], work_v.at[3])

        @pl.loop(0, HRPT)
        def _(i):
          work_v[0, i, :] = work_v[3, i, :] * (
              work_v[0, i, :] + work_v[1, i, :] + work_v[2, i, :])

        pltpu.sync_copy(work_v.at[0], u_spm.at[hslab])

        @pl.when(cid == 0)
        def _():
          pltpu.sync_copy(work_v.at[0], u_out.at[hslab])
    else:
      pltpu.sync_copy(u_hbm.at[slab], u_spm.at[slab])

    _zero_acc_slab(rows_v, acc, sid)
    ds_src.wait()
    ds_dst.wait()
    plsc.subcore_barrier()
    _sweep_pipelined(u_spm, src_v, dst_v, rows_v, acc, sem_g, sem_s)
    plsc.subcore_barrier()
    pltpu.sync_copy(acc.at[slab], agg_out.at[cid, slab])

  return sc_hop


_sc_hop = _make_sc_hop(combine=False)
_sc_hop_fused = _make_sc_hop(combine=True)


def _row_specs(n):
  return [pl.BlockSpec((BN, L), lambda i: (i, 0)) for _ in range(n)]


def _tc_pre(x, wp, d0, d1):
  def body(x_ref, w_ref, d0_ref, d1_ref, u0_ref, inv_ref, dis_ref):
    deg = d0_ref[...] + d1_ref[...] + 1.0
    inv = 1.0 / deg
    dis = lax.rsqrt(deg)
    z = jnp.dot(x_ref[...], w_ref[...], preferred_element_type=jnp.float32)
    u0_ref[...] = dis * z
    inv_ref[...] = inv
    dis_ref[...] = dis

  return pl.pallas_call(
      body,
      grid=(N_PAD // BN,),
      in_specs=[pl.BlockSpec((BN, D), lambda i: (i, 0)),
                pl.BlockSpec((D, L), lambda i: (0, 0))] + _row_specs(2),
      out_specs=_row_specs(3),
      out_shape=[jax.ShapeDtypeStruct((N_PAD, L), jnp.float32)] * 3,
  )(x, wp, d0, d1)


def _tc_post(a0, a1, u1, dis, b16):
  def body(a0_ref, a1_ref, u1_ref, dis_ref, b_ref, o_ref):
    h2 = dis_ref[...] * (a0_ref[...] + a1_ref[...] + u1_ref[...])
    logits = h2 + b_ref[...]
    col = lax.broadcasted_iota(jnp.int32, (BN, L), 1)
    valid = col < C
    masked = jnp.where(valid, logits, jnp.float32(-1e30))
    m = jnp.max(masked, axis=1, keepdims=True)
    s = jnp.sum(jnp.where(valid, jnp.exp(logits - m), 0.0), axis=1,
                keepdims=True)
    o_ref[...] = logits - m - jnp.log(s)

  return pl.pallas_call(
      body,
      grid=(N_PAD // BN,),
      in_specs=_row_specs(3) + [pl.BlockSpec((BN, L), lambda i: (i, 0)),
                                pl.BlockSpec((1, L), lambda i: (0, 0))],
      out_specs=_row_specs(1)[0],
      out_shape=jax.ShapeDtypeStruct((N_PAD, L), jnp.float32),
  )(a0, a1, u1, dis, b16)


def kernel(x, edge_index, W, b):
  wp = jnp.pad(W.T.astype(jnp.float32), ((0, 0), (0, L - C)))
  b16 = jnp.pad(b.astype(jnp.float32), (0, L - C)).reshape(1, L)

  degp = _sc_deg(edge_index)
  u0, inv, dis = _tc_pre(x, wp, degp[0], degp[1])
  a1 = _sc_hop(u0, edge_index)
  a2, u1 = _sc_hop_fused(a1[0], a1[1], u0, inv, edge_index)
  out = _tc_post(a2[0], a2[1], u1, dis, b16)
  return out[:N, :C]
